# baseline (speedup 1.0000x reference)
import functools
import math

import jax
import jax.numpy as jnp
from jax import lax
from jax.experimental import pallas as pl
from jax.experimental.pallas import tpu as pltpu
from jax.experimental.pallas import tpu_sc as plsc

NW = 32


_NBUF = 4


@functools.lru_cache(maxsize=None)
def _mk_ring_gather(e_pad, n_outer, chunk, d, dtype_name):
    dtype = jnp.dtype(dtype_name)
    per_w = n_outer * _NBUF * chunk
    mesh = plsc.VectorSubcoreMesh(core_axis_name="c", subcore_axis_name="s")
    if d == 1:
        out_sd = jax.ShapeDtypeStruct((e_pad,), dtype)
        row_t = pltpu.VMEM((chunk,), dtype)
    else:
        out_sd = jax.ShapeDtypeStruct((e_pad, d), dtype)
        row_t = pltpu.VMEM((chunk, d), dtype)
    scratch = []
    for _ in range(_NBUF):
        scratch += [
            pltpu.VMEM((chunk,), jnp.int32),
            row_t,
            pltpu.SemaphoreType.DMA,
            pltpu.SemaphoreType.DMA,
        ]

    @functools.partial(
        pl.kernel,
        mesh=mesh,
        out_type=out_sd,
        scratch_types=scratch,
    )
    def gather_k(table_hbm, idx_hbm, out_hbm, *bufs):
        wid = lax.axis_index("s") * 2 + lax.axis_index("c")
        base = wid * per_w

        def body(jo, carry):
            start0 = base + jo * (_NBUF * chunk)
            for b in range(_NBUF):
                idx_v, rows_v, sem_g, _ = bufs[4 * b : 4 * b + 4]
                st = start0 + b * chunk
                pltpu.sync_copy(idx_hbm.at[pl.ds(st, chunk)], idx_v)
                pltpu.async_copy(table_hbm.at[idx_v], rows_v, sem_g)
            for b in range(_NBUF):
                idx_v, rows_v, sem_g, sem_o = bufs[4 * b : 4 * b + 4]
                st = start0 + b * chunk
                pltpu.make_async_copy(table_hbm.at[idx_v], rows_v, sem_g).wait()
                pltpu.async_copy(rows_v, out_hbm.at[pl.ds(st, chunk)], sem_o)
            for b in range(_NBUF):
                idx_v, rows_v, _, sem_o = bufs[4 * b : 4 * b + 4]
                st = start0 + b * chunk
                pltpu.make_async_copy(
                    rows_v, out_hbm.at[pl.ds(st, chunk)], sem_o
                ).wait()
            return carry

        lax.fori_loop(0, n_outer, body, 0)

    return gather_k


def _sc_gather(table, idx):
    e = idx.shape[0]
    rows = table.shape[0]
    d = 1 if table.ndim == 1 else table.shape[1]
    chunk = 2048 if d == 1 else 240
    n_outer = -(-e // (NW * _NBUF * chunk))
    e_pad = n_outer * _NBUF * chunk * NW
    pad = jnp.arange(e_pad - e, dtype=jnp.int32) % jnp.int32(rows)
    idx_p = jnp.concatenate([idx.astype(jnp.int32), pad])
    k = _mk_ring_gather(e_pad, n_outer, chunk, d, str(table.dtype))
    return k(table, idx_p)[:e]


def _gtab(table, idx):
    rows = table.shape[0]
    idx_c = jnp.clip(idx, 0, rows - 1)
    return _sc_gather(table, idx_c)


def _grow3(table, idx):
    rows = table.shape[0]
    idx_c = jnp.clip(idx, 0, rows - 1)
    cols = [_sc_gather(jnp.reshape(table[:, j], (rows,)), idx_c) for j in range(3)]
    return jnp.stack(cols, axis=1)


def _gcn_v(x, src, dst, W, b):
    N = x.shape[0]
    h = x @ W
    loop = jnp.arange(N, dtype=src.dtype)
    s = jnp.concatenate([src, loop]); d = jnp.concatenate([dst, loop])
    deg = jnp.zeros((N,), jnp.float32).at[d].add(1.0)
    dis = jnp.where(deg > 0, 1.0 / jnp.sqrt(deg), 0.0)
    norm = _gtab(dis, s) * _gtab(dis, d)
    if W.shape[1] >= 8:
        hs = _gtab(h, s)
    else:
        hs = _grow3(h, s)
    out = jnp.zeros((N, W.shape[1]), jnp.float32).at[d].add(hs * norm[:, None])
    return out + b


def _gat_v(x, src, dst, W, att_s, att_d, b):
    N = x.shape[0]
    h = x @ W
    loop = jnp.arange(N, dtype=src.dtype)
    s = jnp.concatenate([src, loop]); d = jnp.concatenate([dst, loop])
    e = _gtab(h @ att_s, s) + _gtab(h @ att_d, d)
    e = jnp.where(e > 0, e, 0.2 * e)
    emax = jnp.full((N,), -jnp.inf, jnp.float32).at[d].max(e)
    a = jnp.exp(e - _gtab(emax, d))
    den = jnp.zeros((N,), jnp.float32).at[d].add(a)
    a = a / _gtab(den, d)
    hs = _gtab(h, s)
    out = jnp.zeros((N, W.shape[1]), jnp.float32).at[d].add(hs * a[:, None])
    return out + b


def _pool_v(h, src, dst, p):
    N = h.shape[0]
    k = int(math.ceil(0.5 * N))
    score = jnp.tanh((h @ p) / jnp.linalg.norm(p))
    _, perm = jax.lax.top_k(score, k)
    x_new = h[perm] * score[perm][:, None]
    mask = jnp.zeros((N,), dtype=bool).at[perm].set(True)
    new_idx = jnp.zeros((N,), dtype=jnp.int32).at[perm].set(jnp.arange(k, dtype=jnp.int32))
    valid = (src < N) & (dst < N)
    mi = jnp.where(mask, new_idx, jnp.int32(-1))
    rs = _gtab(mi, src)
    rd = _gtab(mi, dst)
    em = valid & (rs >= 0) & (rd >= 0)
    s2 = jnp.where(em, rs, jnp.int32(k))
    d2 = jnp.where(em, rd, jnp.int32(k))
    batch = jnp.zeros((k,), jnp.int32)
    return x_new, s2, d2, batch, perm


def kernel(x, edge_index, W_enc0, b_enc0, p0, W_dec0, b_dec0, W_enc1, as1, ad1, b_enc1, p1, W_dec1, b_dec1, W_enc2, as2, ad2, b_enc2, p2, W_dec2, b_dec2):
    src = edge_index[0].astype(jnp.int32); dst = edge_index[1].astype(jnp.int32)
    outputs = []; batches = []
    h = _gcn_v(x, src, dst, W_enc0, b_enc0)
    h, src, dst, batch, _ = _pool_v(h, src, dst, p0)
    h = _gcn_v(h, src, dst, W_dec0, b_dec0)
    outputs.append(h); batches.append(batch)
    h = _gat_v(h, src, dst, W_enc1, as1, ad1, b_enc1)
    h, src, dst, batch, _ = _pool_v(h, src, dst, p1)
    h = _gcn_v(h, src, dst, W_dec1, b_dec1)
    outputs.append(h); batches.append(batch)
    h = _gat_v(h, src, dst, W_enc2, as2, ad2, b_enc2)
    h, src, dst, batch, _ = _pool_v(h, src, dst, p2)
    h = _gcn_v(h, src, dst, W_dec2, b_dec2)
    outputs.append(h); batches.append(batch)
    return (outputs[0], outputs[1], outputs[2], batches[0], batches[1], batches[2])


# speedup vs baseline: 1.6719x; 1.6719x over previous
import functools
import math

import jax
import jax.numpy as jnp
from jax import lax
from jax.experimental import pallas as pl
from jax.experimental.pallas import tpu as pltpu
from jax.experimental.pallas import tpu_sc as plsc

NW = 32


_NBUF = 4


@functools.lru_cache(maxsize=None)
def _mk_ring_gather(e_pad, n_outer, chunk, d, dtype_name, staged):
    dtype = jnp.dtype(dtype_name)
    per_w = n_outer * _NBUF * chunk
    mesh = plsc.VectorSubcoreMesh(core_axis_name="c", subcore_axis_name="s")
    if d == 1:
        out_sd = jax.ShapeDtypeStruct((e_pad,), dtype)
        row_t = pltpu.VMEM((chunk,), dtype)
    else:
        out_sd = jax.ShapeDtypeStruct((e_pad, d), dtype)
        row_t = pltpu.VMEM((chunk, d), dtype)
    scratch = []
    for _ in range(_NBUF):
        scratch += [
            pltpu.VMEM((chunk,), jnp.int32),
            row_t,
            pltpu.SemaphoreType.DMA,
            pltpu.SemaphoreType.DMA,
        ]
    if staged:
        scratch.append(pltpu.VMEM_SHARED((staged,), dtype))

    @functools.partial(
        pl.kernel,
        mesh=mesh,
        out_type=out_sd,
        scratch_types=scratch,
    )
    def gather_k(table_hbm, idx_hbm, out_hbm, *bufs):
        wid = lax.axis_index("s") * 2 + lax.axis_index("c")
        base = wid * per_w
        if staged:
            table_ref = bufs[-1]
            bufs = bufs[:-1]

            @pl.when(lax.axis_index("s") == 0)
            def _stage():
                pltpu.sync_copy(table_hbm, table_ref)

            plsc.subcore_barrier()
        else:
            table_ref = table_hbm

        def body(jo, carry):
            start0 = base + jo * (_NBUF * chunk)
            for b in range(_NBUF):
                idx_v, rows_v, sem_g, _ = bufs[4 * b : 4 * b + 4]
                st = start0 + b * chunk
                pltpu.sync_copy(idx_hbm.at[pl.ds(st, chunk)], idx_v)
                pltpu.async_copy(table_ref.at[idx_v], rows_v, sem_g)
            for b in range(_NBUF):
                idx_v, rows_v, sem_g, sem_o = bufs[4 * b : 4 * b + 4]
                st = start0 + b * chunk
                pltpu.make_async_copy(table_ref.at[idx_v], rows_v, sem_g).wait()
                pltpu.async_copy(rows_v, out_hbm.at[pl.ds(st, chunk)], sem_o)
            for b in range(_NBUF):
                idx_v, rows_v, _, sem_o = bufs[4 * b : 4 * b + 4]
                st = start0 + b * chunk
                pltpu.make_async_copy(
                    rows_v, out_hbm.at[pl.ds(st, chunk)], sem_o
                ).wait()
            return carry

        lax.fori_loop(0, n_outer, body, 0)

    return gather_k


def _sc_gather(table, idx):
    e = idx.shape[0]
    rows = table.shape[0]
    d = 1 if table.ndim == 1 else table.shape[1]
    chunk = 2048 if d == 1 else 240
    n_outer = -(-e // (NW * _NBUF * chunk))
    e_pad = n_outer * _NBUF * chunk * NW
    pad = jnp.arange(e_pad - e, dtype=jnp.int32) % jnp.int32(rows)
    idx_p = jnp.concatenate([idx.astype(jnp.int32), pad])
    staged = rows if (d == 1 and rows * 4 <= 6 * 1024 * 1024) else 0
    k = _mk_ring_gather(e_pad, n_outer, chunk, d, str(table.dtype), staged)
    return k(table, idx_p)[:e]


def _gtab(table, idx):
    rows = table.shape[0]
    idx_c = jnp.clip(idx, 0, rows - 1)
    return _sc_gather(table, idx_c)


def _grow3(table, idx):
    rows = table.shape[0]
    idx_c = jnp.clip(idx, 0, rows - 1)
    cols = [_sc_gather(jnp.reshape(table[:, j], (rows,)), idx_c) for j in range(3)]
    return jnp.stack(cols, axis=1)


def _gcn_v(x, src, dst, W, b):
    N = x.shape[0]
    h = x @ W
    loop = jnp.arange(N, dtype=src.dtype)
    s = jnp.concatenate([src, loop]); d = jnp.concatenate([dst, loop])
    deg = jnp.zeros((N,), jnp.float32).at[d].add(1.0)
    dis = jnp.where(deg > 0, 1.0 / jnp.sqrt(deg), 0.0)
    norm = _gtab(dis, s) * _gtab(dis, d)
    if W.shape[1] >= 8:
        hs = _gtab(h, s)
    else:
        hs = _grow3(h, s)
    out = jnp.zeros((N, W.shape[1]), jnp.float32).at[d].add(hs * norm[:, None])
    return out + b


def _gat_v(x, src, dst, W, att_s, att_d, b):
    N = x.shape[0]
    h = x @ W
    loop = jnp.arange(N, dtype=src.dtype)
    s = jnp.concatenate([src, loop]); d = jnp.concatenate([dst, loop])
    e = _gtab(h @ att_s, s) + _gtab(h @ att_d, d)
    e = jnp.where(e > 0, e, 0.2 * e)
    emax = jnp.full((N,), -jnp.inf, jnp.float32).at[d].max(e)
    a = jnp.exp(e - _gtab(emax, d))
    den = jnp.zeros((N,), jnp.float32).at[d].add(a)
    a = a / _gtab(den, d)
    hs = _gtab(h, s)
    out = jnp.zeros((N, W.shape[1]), jnp.float32).at[d].add(hs * a[:, None])
    return out + b


def _pool_v(h, src, dst, p):
    N = h.shape[0]
    k = int(math.ceil(0.5 * N))
    score = jnp.tanh((h @ p) / jnp.linalg.norm(p))
    _, perm = jax.lax.top_k(score, k)
    x_new = h[perm] * score[perm][:, None]
    mask = jnp.zeros((N,), dtype=bool).at[perm].set(True)
    new_idx = jnp.zeros((N,), dtype=jnp.int32).at[perm].set(jnp.arange(k, dtype=jnp.int32))
    valid = (src < N) & (dst < N)
    mi = jnp.where(mask, new_idx, jnp.int32(-1))
    rs = _gtab(mi, src)
    rd = _gtab(mi, dst)
    em = valid & (rs >= 0) & (rd >= 0)
    s2 = jnp.where(em, rs, jnp.int32(k))
    d2 = jnp.where(em, rd, jnp.int32(k))
    batch = jnp.zeros((k,), jnp.int32)
    return x_new, s2, d2, batch, perm


def kernel(x, edge_index, W_enc0, b_enc0, p0, W_dec0, b_dec0, W_enc1, as1, ad1, b_enc1, p1, W_dec1, b_dec1, W_enc2, as2, ad2, b_enc2, p2, W_dec2, b_dec2):
    src = edge_index[0].astype(jnp.int32); dst = edge_index[1].astype(jnp.int32)
    outputs = []; batches = []
    h = _gcn_v(x, src, dst, W_enc0, b_enc0)
    h, src, dst, batch, _ = _pool_v(h, src, dst, p0)
    h = _gcn_v(h, src, dst, W_dec0, b_dec0)
    outputs.append(h); batches.append(batch)
    h = _gat_v(h, src, dst, W_enc1, as1, ad1, b_enc1)
    h, src, dst, batch, _ = _pool_v(h, src, dst, p1)
    h = _gcn_v(h, src, dst, W_dec1, b_dec1)
    outputs.append(h); batches.append(batch)
    h = _gat_v(h, src, dst, W_enc2, as2, ad2, b_enc2)
    h, src, dst, batch, _ = _pool_v(h, src, dst, p2)
    h = _gcn_v(h, src, dst, W_dec2, b_dec2)
    outputs.append(h); batches.append(batch)
    return (outputs[0], outputs[1], outputs[2], batches[0], batches[1], batches[2])


# speedup vs baseline: 3.1515x; 1.8850x over previous
import functools
import math

import jax
import jax.numpy as jnp
from jax import lax
from jax.experimental import pallas as pl
from jax.experimental.pallas import tpu as pltpu
from jax.experimental.pallas import tpu_sc as plsc

NW = 32


_NBUF = 4


@functools.lru_cache(maxsize=None)
def _mk_ring_gather(e_pad, n_outer, chunk, d, dtype_name, staged):
    dtype = jnp.dtype(dtype_name)
    per_w = n_outer * _NBUF * chunk
    mesh = plsc.VectorSubcoreMesh(core_axis_name="c", subcore_axis_name="s")
    if d == 1:
        out_sd = jax.ShapeDtypeStruct((e_pad,), dtype)
        row_t = pltpu.VMEM((chunk,), dtype)
    else:
        out_sd = jax.ShapeDtypeStruct((e_pad, d), dtype)
        row_t = pltpu.VMEM((chunk, d), dtype)
    scratch = []
    for _ in range(_NBUF):
        scratch += [
            pltpu.VMEM((chunk,), jnp.int32),
            row_t,
            pltpu.SemaphoreType.DMA,
            pltpu.SemaphoreType.DMA,
        ]
    if staged:
        scratch.append(pltpu.VMEM_SHARED((staged,), dtype))

    @functools.partial(
        pl.kernel,
        mesh=mesh,
        out_type=out_sd,
        scratch_types=scratch,
    )
    def gather_k(table_hbm, idx_hbm, out_hbm, *bufs):
        wid = lax.axis_index("s") * 2 + lax.axis_index("c")
        base = wid * per_w
        if staged:
            table_ref = bufs[-1]
            bufs = bufs[:-1]

            @pl.when(lax.axis_index("s") == 0)
            def _stage():
                pltpu.sync_copy(table_hbm, table_ref)

            plsc.subcore_barrier()
        else:
            table_ref = table_hbm

        def body(jo, carry):
            start0 = base + jo * (_NBUF * chunk)
            for b in range(_NBUF):
                idx_v, rows_v, sem_g, _ = bufs[4 * b : 4 * b + 4]
                st = start0 + b * chunk
                pltpu.sync_copy(idx_hbm.at[pl.ds(st, chunk)], idx_v)
                pltpu.async_copy(table_ref.at[idx_v], rows_v, sem_g)
            for b in range(_NBUF):
                idx_v, rows_v, sem_g, sem_o = bufs[4 * b : 4 * b + 4]
                st = start0 + b * chunk
                pltpu.make_async_copy(table_ref.at[idx_v], rows_v, sem_g).wait()
                pltpu.async_copy(rows_v, out_hbm.at[pl.ds(st, chunk)], sem_o)
            for b in range(_NBUF):
                idx_v, rows_v, _, sem_o = bufs[4 * b : 4 * b + 4]
                st = start0 + b * chunk
                pltpu.make_async_copy(
                    rows_v, out_hbm.at[pl.ds(st, chunk)], sem_o
                ).wait()
            return carry

        lax.fori_loop(0, n_outer, body, 0)

    return gather_k


def _sc_gather(table, idx):
    e = idx.shape[0]
    rows = table.shape[0]
    d = 1 if table.ndim == 1 else table.shape[1]
    chunk = 2048 if d == 1 else 240
    n_outer = -(-e // (NW * _NBUF * chunk))
    e_pad = n_outer * _NBUF * chunk * NW
    pad = jnp.arange(e_pad - e, dtype=jnp.int32) % jnp.int32(rows)
    idx_p = jnp.concatenate([idx.astype(jnp.int32), pad])
    staged = rows if (d == 1 and rows * 4 <= 6 * 1024 * 1024) else 0
    k = _mk_ring_gather(e_pad, n_outer, chunk, d, str(table.dtype), staged)
    return k(table, idx_p)[:e]


def _gtab(table, idx):
    rows = table.shape[0]
    idx_c = jnp.clip(idx, 0, rows - 1)
    return _sc_gather(table, idx_c)


def _grow3(table, idx):
    rows = table.shape[0]
    idx_c = jnp.clip(idx, 0, rows - 1)
    cols = [_sc_gather(jnp.reshape(table[:, j], (rows,)), idx_c) for j in range(3)]
    return jnp.stack(cols, axis=1)


def _gcn_v(x, src, dst, W, b):
    N = x.shape[0]
    h = x @ W
    loop = jnp.arange(N, dtype=src.dtype)
    s = jnp.concatenate([src, loop]); d = jnp.concatenate([dst, loop])
    deg = jnp.zeros((N,), jnp.float32).at[d].add(1.0)
    dis = jnp.where(deg > 0, 1.0 / jnp.sqrt(deg), 0.0)
    norm = _gtab(dis, s) * _gtab(dis, d)
    if W.shape[0] <= 4:
        hs = _grow3(x, s) @ W
    else:
        hs = _grow3(h, s)
    out = jnp.zeros((N, W.shape[1]), jnp.float32).at[d].add(hs * norm[:, None])
    return out + b


def _gat_v(x, src, dst, W, att_s, att_d, b):
    N = x.shape[0]
    h = x @ W
    loop = jnp.arange(N, dtype=src.dtype)
    s = jnp.concatenate([src, loop]); d = jnp.concatenate([dst, loop])
    e = _gtab(h @ att_s, s) + _gtab(h @ att_d, d)
    e = jnp.where(e > 0, e, 0.2 * e)
    emax = jnp.full((N,), -jnp.inf, jnp.float32).at[d].max(e)
    a = jnp.exp(e - _gtab(emax, d))
    den = jnp.zeros((N,), jnp.float32).at[d].add(a)
    a = a / _gtab(den, d)
    hs = _grow3(x, s) @ W
    out = jnp.zeros((N, W.shape[1]), jnp.float32).at[d].add(hs * a[:, None])
    return out + b


def _pool_v(h, src, dst, p):
    N = h.shape[0]
    k = int(math.ceil(0.5 * N))
    score = jnp.tanh((h @ p) / jnp.linalg.norm(p))
    _, perm = jax.lax.top_k(score, k)
    x_new = h[perm] * score[perm][:, None]
    mask = jnp.zeros((N,), dtype=bool).at[perm].set(True)
    new_idx = jnp.zeros((N,), dtype=jnp.int32).at[perm].set(jnp.arange(k, dtype=jnp.int32))
    valid = (src < N) & (dst < N)
    mi = jnp.where(mask, new_idx, jnp.int32(-1))
    rs = _gtab(mi, src)
    rd = _gtab(mi, dst)
    em = valid & (rs >= 0) & (rd >= 0)
    s2 = jnp.where(em, rs, jnp.int32(k))
    d2 = jnp.where(em, rd, jnp.int32(k))
    batch = jnp.zeros((k,), jnp.int32)
    return x_new, s2, d2, batch, perm


def kernel(x, edge_index, W_enc0, b_enc0, p0, W_dec0, b_dec0, W_enc1, as1, ad1, b_enc1, p1, W_dec1, b_dec1, W_enc2, as2, ad2, b_enc2, p2, W_dec2, b_dec2):
    src = edge_index[0].astype(jnp.int32); dst = edge_index[1].astype(jnp.int32)
    outputs = []; batches = []
    h = _gcn_v(x, src, dst, W_enc0, b_enc0)
    h, src, dst, batch, _ = _pool_v(h, src, dst, p0)
    h = _gcn_v(h, src, dst, W_dec0, b_dec0)
    outputs.append(h); batches.append(batch)
    h = _gat_v(h, src, dst, W_enc1, as1, ad1, b_enc1)
    h, src, dst, batch, _ = _pool_v(h, src, dst, p1)
    h = _gcn_v(h, src, dst, W_dec1, b_dec1)
    outputs.append(h); batches.append(batch)
    h = _gat_v(h, src, dst, W_enc2, as2, ad2, b_enc2)
    h, src, dst, batch, _ = _pool_v(h, src, dst, p2)
    h = _gcn_v(h, src, dst, W_dec2, b_dec2)
    outputs.append(h); batches.append(batch)
    return (outputs[0], outputs[1], outputs[2], batches[0], batches[1], batches[2])


# speedup vs baseline: 3.8993x; 1.2373x over previous
import functools
import math

import jax
import jax.numpy as jnp
from jax import lax
from jax.experimental import pallas as pl
from jax.experimental.pallas import tpu as pltpu
from jax.experimental.pallas import tpu_sc as plsc

NW = 32


_NBUF = 4


@functools.lru_cache(maxsize=None)
def _mk_ring_gather(e_pad, n_outer, chunk, d, dtype_name, staged):
    dtype = jnp.dtype(dtype_name)
    per_w = n_outer * _NBUF * chunk
    mesh = plsc.VectorSubcoreMesh(core_axis_name="c", subcore_axis_name="s")
    if d == 1:
        out_sd = jax.ShapeDtypeStruct((e_pad,), dtype)
        row_t = pltpu.VMEM((chunk,), dtype)
    else:
        out_sd = jax.ShapeDtypeStruct((e_pad, d), dtype)
        row_t = pltpu.VMEM((chunk, d), dtype)
    scratch = []
    for _ in range(_NBUF):
        scratch += [
            pltpu.VMEM((chunk,), jnp.int32),
            row_t,
            pltpu.SemaphoreType.DMA,
            pltpu.SemaphoreType.DMA,
        ]
    if staged:
        scratch.append(pltpu.VMEM_SHARED((staged,), dtype))

    @functools.partial(
        pl.kernel,
        mesh=mesh,
        out_type=out_sd,
        scratch_types=scratch,
    )
    def gather_k(table_hbm, idx_hbm, out_hbm, *bufs):
        wid = lax.axis_index("s") * 2 + lax.axis_index("c")
        base = wid * per_w
        if staged:
            table_ref = bufs[-1]
            bufs = bufs[:-1]

            @pl.when(lax.axis_index("s") == 0)
            def _stage():
                pltpu.sync_copy(table_hbm, table_ref)

            plsc.subcore_barrier()
        else:
            table_ref = table_hbm

        def body(jo, carry):
            start0 = base + jo * (_NBUF * chunk)
            for b in range(_NBUF):
                idx_v, rows_v, sem_g, _ = bufs[4 * b : 4 * b + 4]
                st = start0 + b * chunk
                pltpu.sync_copy(idx_hbm.at[pl.ds(st, chunk)], idx_v)
                pltpu.async_copy(table_ref.at[idx_v], rows_v, sem_g)
            for b in range(_NBUF):
                idx_v, rows_v, sem_g, sem_o = bufs[4 * b : 4 * b + 4]
                st = start0 + b * chunk
                pltpu.make_async_copy(table_ref.at[idx_v], rows_v, sem_g).wait()
                pltpu.async_copy(rows_v, out_hbm.at[pl.ds(st, chunk)], sem_o)
            for b in range(_NBUF):
                idx_v, rows_v, _, sem_o = bufs[4 * b : 4 * b + 4]
                st = start0 + b * chunk
                pltpu.make_async_copy(
                    rows_v, out_hbm.at[pl.ds(st, chunk)], sem_o
                ).wait()
            return carry

        lax.fori_loop(0, n_outer, body, 0)

    return gather_k


@functools.lru_cache(maxsize=None)
def _mk_deg(e_pad, n_chunks, chunk, rows_pad):
    per_w = n_chunks * chunk
    mesh = plsc.VectorSubcoreMesh(core_axis_name="c", subcore_axis_name="s")

    @functools.partial(
        pl.kernel,
        mesh=mesh,
        out_type=jax.ShapeDtypeStruct((2, rows_pad), jnp.float32),
        scratch_types=[
            pltpu.VMEM((chunk,), jnp.int32),
            pltpu.VMEM((chunk,), jnp.float32),
            pltpu.VMEM_SHARED((rows_pad,), jnp.float32),
        ],
    )
    def deg_k(zeros_hbm, ones_hbm, idx_hbm, out_hbm, idx_v, ones_v, acc):
        cid = lax.axis_index("c")
        sid = lax.axis_index("s")
        wid = sid * 2 + cid
        base = wid * per_w
        pltpu.sync_copy(ones_hbm, ones_v)

        @pl.when(sid == 0)
        def _zero():
            pltpu.sync_copy(zeros_hbm, acc)

        plsc.subcore_barrier()

        def body(j, carry):
            start = base + j * chunk
            pltpu.sync_copy(idx_hbm.at[pl.ds(start, chunk)], idx_v)
            pltpu.sync_copy(ones_v, acc.at[idx_v], add=True)
            return carry

        lax.fori_loop(0, n_chunks, body, 0)
        plsc.subcore_barrier()

        @pl.when(sid == 0)
        def _out():
            pltpu.sync_copy(acc, out_hbm.at[cid])

    return deg_k


def _sc_deg(d, n):
    e = d.shape[0]
    chunk = 2048
    n_chunks = -(-e // (NW * chunk))
    e_pad = n_chunks * chunk * NW
    rows_pad = n + 2048
    spread = jnp.arange(e, dtype=jnp.int32) & jnp.int32(2047)
    d_sp = jnp.where(d >= n, jnp.int32(n) + spread, d)
    pad = jnp.int32(n) + (jnp.arange(e_pad - e, dtype=jnp.int32) & jnp.int32(2047))
    idx_p = jnp.concatenate([d_sp, pad])
    k = _mk_deg(e_pad, n_chunks, chunk, rows_pad)
    zeros = jnp.zeros((rows_pad,), jnp.float32)
    ones = jnp.ones((chunk,), jnp.float32)
    parts = k(zeros, ones, idx_p)
    return (parts[0] + parts[1])[:n]


def _sc_gather(table, idx):
    e = idx.shape[0]
    rows = table.shape[0]
    d = 1 if table.ndim == 1 else table.shape[1]
    chunk = 2048 if d == 1 else 240
    n_outer = -(-e // (NW * _NBUF * chunk))
    e_pad = n_outer * _NBUF * chunk * NW
    pad = jnp.arange(e_pad - e, dtype=jnp.int32) % jnp.int32(rows)
    idx_p = jnp.concatenate([idx.astype(jnp.int32), pad])
    staged = rows if (d == 1 and rows * 4 <= 6 * 1024 * 1024) else 0
    k = _mk_ring_gather(e_pad, n_outer, chunk, d, str(table.dtype), staged)
    return k(table, idx_p)[:e]


def _gtab(table, idx):
    rows = table.shape[0]
    idx_c = jnp.clip(idx, 0, rows - 1)
    return _sc_gather(table, idx_c)


def _grow3(table, idx):
    rows = table.shape[0]
    idx_c = jnp.clip(idx, 0, rows - 1)
    cols = [_sc_gather(jnp.reshape(table[:, j], (rows,)), idx_c) for j in range(3)]
    return jnp.stack(cols, axis=1)


def _gcn_v(x, src, dst, W, b):
    N = x.shape[0]
    h = x @ W
    loop = jnp.arange(N, dtype=src.dtype)
    s = jnp.concatenate([src, loop]); d = jnp.concatenate([dst, loop])
    deg = _sc_deg(dst, N) + 1.0
    dis = jnp.where(deg > 0, 1.0 / jnp.sqrt(deg), 0.0)
    norm = _gtab(dis, s) * _gtab(dis, d)
    if W.shape[0] <= 4:
        hs = _grow3(x, s) @ W
    else:
        hs = _grow3(h, s)
    out = jnp.zeros((N, W.shape[1]), jnp.float32).at[d].add(hs * norm[:, None])
    return out + b


def _gat_v(x, src, dst, W, att_s, att_d, b):
    N = x.shape[0]
    h = x @ W
    loop = jnp.arange(N, dtype=src.dtype)
    s = jnp.concatenate([src, loop]); d = jnp.concatenate([dst, loop])
    e = _gtab(h @ att_s, s) + _gtab(h @ att_d, d)
    e = jnp.where(e > 0, e, 0.2 * e)
    emax = jnp.full((N,), -jnp.inf, jnp.float32).at[d].max(e)
    a = jnp.exp(e - _gtab(emax, d))
    den = jnp.zeros((N,), jnp.float32).at[d].add(a)
    a = a / _gtab(den, d)
    hs = _grow3(x, s) @ W
    out = jnp.zeros((N, W.shape[1]), jnp.float32).at[d].add(hs * a[:, None])
    return out + b


def _pool_v(h, src, dst, p):
    N = h.shape[0]
    k = int(math.ceil(0.5 * N))
    score = jnp.tanh((h @ p) / jnp.linalg.norm(p))
    _, perm = jax.lax.top_k(score, k)
    x_new = h[perm] * score[perm][:, None]
    mask = jnp.zeros((N,), dtype=bool).at[perm].set(True)
    new_idx = jnp.zeros((N,), dtype=jnp.int32).at[perm].set(jnp.arange(k, dtype=jnp.int32))
    valid = (src < N) & (dst < N)
    mi = jnp.where(mask, new_idx, jnp.int32(-1))
    rs = _gtab(mi, src)
    rd = _gtab(mi, dst)
    em = valid & (rs >= 0) & (rd >= 0)
    s2 = jnp.where(em, rs, jnp.int32(k))
    d2 = jnp.where(em, rd, jnp.int32(k))
    batch = jnp.zeros((k,), jnp.int32)
    return x_new, s2, d2, batch, perm


def kernel(x, edge_index, W_enc0, b_enc0, p0, W_dec0, b_dec0, W_enc1, as1, ad1, b_enc1, p1, W_dec1, b_dec1, W_enc2, as2, ad2, b_enc2, p2, W_dec2, b_dec2):
    src = edge_index[0].astype(jnp.int32); dst = edge_index[1].astype(jnp.int32)
    outputs = []; batches = []
    h = _gcn_v(x, src, dst, W_enc0, b_enc0)
    h, src, dst, batch, _ = _pool_v(h, src, dst, p0)
    h = _gcn_v(h, src, dst, W_dec0, b_dec0)
    outputs.append(h); batches.append(batch)
    h = _gat_v(h, src, dst, W_enc1, as1, ad1, b_enc1)
    h, src, dst, batch, _ = _pool_v(h, src, dst, p1)
    h = _gcn_v(h, src, dst, W_dec1, b_dec1)
    outputs.append(h); batches.append(batch)
    h = _gat_v(h, src, dst, W_enc2, as2, ad2, b_enc2)
    h, src, dst, batch, _ = _pool_v(h, src, dst, p2)
    h = _gcn_v(h, src, dst, W_dec2, b_dec2)
    outputs.append(h); batches.append(batch)
    return (outputs[0], outputs[1], outputs[2], batches[0], batches[1], batches[2])
